# Initial kernel scaffold; baseline (speedup 1.0000x reference)
import jax
import jax.numpy as jnp
from jax.experimental import pallas as pl


def kernel(node_features, edge_index, W, correct_bias, incorrect_bias, Q):
    raise NotImplementedError("write your pallas kernel here")


# speedup vs baseline: 7.7278x; 7.7278x over previous
import functools

import jax
import jax.numpy as jnp
from jax import lax
from jax.experimental import pallas as pl
from jax.experimental.pallas import tpu as pltpu
from jax.experimental.pallas import tpu_sc as plsc

EMB = 128
NQ = 5000
R = 5120
NC = 2
NS = 16
NW = NC * NS
CH = 128
BLK = 512


def _sc_aggregate(node_features, src2d, dst2d):
    ncht = src2d.shape[0]
    nj = (ncht + NW - 1) // NW
    stripe = R // NS

    mesh = plsc.VectorSubcoreMesh(
        core_axis_name="c", subcore_axis_name="s",
        num_cores=NC, num_subcores=NS)

    @functools.partial(
        pl.kernel,
        out_type=(
            jax.ShapeDtypeStruct((NC, R, EMB), jnp.float32),
            jax.ShapeDtypeStruct((NC * R,), jnp.float32),
        ),
        mesh=mesh,
        scratch_types=[
            pltpu.VMEM((1, CH), jnp.int32),
            pltpu.VMEM((1, CH), jnp.int32),
            pltpu.VMEM((CH, EMB), jnp.float32),
            pltpu.VMEM((CH,), jnp.float32),
            pltpu.VMEM_SHARED((R, EMB), jnp.float32),
            pltpu.VMEM_SHARED((R,), jnp.float32),
            pltpu.SemaphoreType.DMA,
        ],
    )
    def k(nf_hbm, src_hbm, dst_hbm, agg_out, deg_out,
          src_v, dst_v, rows_v, ones_v, agg_sh, deg_sh, sem):
        c_idx = lax.axis_index("c")
        s_idx = lax.axis_index("s")
        wid = s_idx * NC + c_idx

        zero16 = jnp.zeros((16,), jnp.float32)
        one16 = jnp.ones((16,), jnp.float32)
        lane16 = lax.iota(jnp.int32, 16)

        def zrow(r, carry):
            row = rows_v.at[r]
            for i in range(EMB // 16):
                row[pl.ds(i * 16, 16)] = zero16
            return carry
        lax.fori_loop(0, CH, zrow, 0)
        for i in range(CH // 16):
            ones_v[pl.ds(i * 16, 16)] = one16

        base = s_idx * stripe
        off = 0
        while off < stripe:
            n = min(CH, stripe - off)
            pltpu.sync_copy(rows_v.at[pl.ds(0, n)],
                            agg_sh.at[pl.ds(base + off, n)])
            off += n
        off = 0
        while off < stripe:
            n = min(EMB, stripe - off)
            pltpu.sync_copy(rows_v.at[0, pl.ds(0, n)],
                            deg_sh.at[pl.ds(base + off, n)])
            off += n
        plsc.subcore_barrier()

        def body(j, carry):
            ch = wid + NW * j

            @pl.when(ch < ncht)
            def _():
                pltpu.sync_copy(src_hbm.at[ch], src_v.at[0])
                pltpu.sync_copy(dst_hbm.at[ch], dst_v.at[0])
                dst_row = dst_v.at[0]
                for i in range(CH // 16):
                    d = dst_row[pl.ds(i * 16, 16)]
                    dc = jnp.where(d < NQ, d, NQ + lane16)
                    dst_row[pl.ds(i * 16, 16)] = dc
                pltpu.async_copy(nf_hbm.at[src_v.at[0]], rows_v, sem).wait()
                pltpu.sync_copy(rows_v, agg_sh.at[dst_v.at[0]], add=True)
                pltpu.sync_copy(ones_v, deg_sh.at[dst_v.at[0]], add=True)
            return carry

        lax.fori_loop(0, nj, body, 0)
        plsc.subcore_barrier()

        pltpu.sync_copy(agg_sh.at[pl.ds(base, stripe)],
                        agg_out.at[c_idx, pl.ds(base, stripe)])

        @pl.when(s_idx == 0)
        def _():
            pltpu.sync_copy(deg_sh, deg_out.at[pl.ds(c_idx * R, R)])

    return k(node_features, src2d, dst2d)


def _tc_dense_body(q_ref, nf_ref, agg_ref, deg_ref, w_ref, cb_ref, ib_ref,
                   wrong_ref, right_ref):
    i = pl.program_id(0)
    agg = agg_ref[0] + agg_ref[1]
    deg = jnp.sum(deg_ref[...], axis=0)
    x = nf_ref[...] + agg / jnp.maximum(deg, 1.0)[:, None]
    h = jnp.maximum(jnp.dot(x, w_ref[...],
                            preferred_element_type=jnp.float32), 0.0)
    rows = i * BLK + lax.broadcasted_iota(jnp.int32, (BLK, EMB), 0)
    base = jnp.where(rows < q_ref[0, 0], h, 0.0)
    wrong_ref[...] = base + ib_ref[...]
    right_ref[...] = base + cb_ref[...]


def _tc_dense(q, node_features, agg_p, deg_p, W, correct_bias, incorrect_bias):
    grid = (NQ + BLK - 1) // BLK
    return pl.pallas_call(
        _tc_dense_body,
        grid=(grid,),
        in_specs=[
            pl.BlockSpec(memory_space=pltpu.SMEM),
            pl.BlockSpec((BLK, EMB), lambda i: (i, 0)),
            pl.BlockSpec((NC, BLK, EMB), lambda i: (0, i, 0)),
            pl.BlockSpec((NC, BLK), lambda i: (0, i)),
            pl.BlockSpec((EMB, EMB), lambda i: (0, 0)),
            pl.BlockSpec((1, EMB), lambda i: (0, 0)),
            pl.BlockSpec((1, EMB), lambda i: (0, 0)),
        ],
        out_specs=[
            pl.BlockSpec((BLK, EMB), lambda i: (i, 0)),
            pl.BlockSpec((BLK, EMB), lambda i: (i, 0)),
        ],
        out_shape=[
            jax.ShapeDtypeStruct((NQ, EMB), jnp.float32),
            jax.ShapeDtypeStruct((NQ, EMB), jnp.float32),
        ],
    )(q, node_features, agg_p, deg_p, W, correct_bias, incorrect_bias)


def kernel(node_features, edge_index, W, correct_bias, incorrect_bias, Q):
    e = edge_index.shape[1]
    src2d = edge_index[0].reshape(e // CH, CH)
    dst2d = edge_index[1].reshape(e // CH, CH)
    agg_p, deg_p = _sc_aggregate(node_features, src2d, dst2d)
    deg_p = deg_p.reshape(NC, R)
    q_arr = jnp.asarray(Q, dtype=jnp.int32).reshape(1, 1)
    wrong, right = _tc_dense(q_arr, node_features, agg_p, deg_p, W,
                             correct_bias, incorrect_bias)
    padding = jnp.zeros((1, EMB), dtype=wrong.dtype)
    return jnp.concatenate([wrong, right, padding], axis=0)
